# baseline (speedup 1.0000x reference)
import jax
import jax.numpy as jnp
from jax.experimental import pallas as pl
from jax.experimental.pallas import tpu as pltpu

Q = 16
C = 100
CH = C // 2
P = 196
D = 768
PB = 16
NFULL = P // PB
TAIL = P - NFULL * PB
QR = Q * PB
CRH = CH * PB


def _body(q_ref, k_ref, dist_ref, idx_ref,
          g8_ref, s2_ref, q2_ref, dt_ref, s2t_ref):
    pp = pl.program_id(0)
    hh = pl.program_id(1)

    @pl.when(jnp.logical_and(pp == 0, hh == 0))
    def _init():
        g8_ref[...] = jnp.zeros_like(g8_ref)
        s2_ref[...] = jnp.zeros_like(s2_ref)
        q2_ref[...] = jnp.zeros_like(q2_ref)

    @pl.when(pp < NFULL)
    def _main():
        qblk = q_ref[...]
        kblk = k_ref[...]
        qr = qblk.reshape(QR, D)
        kr = kblk.reshape(CRH, D)
        qhi = qr.astype(jnp.bfloat16)
        qlo = (qr - qhi.astype(jnp.float32)).astype(jnp.bfloat16)
        khi = kr.astype(jnp.bfloat16)
        klo = (kr - khi.astype(jnp.float32)).astype(jnp.bfloat16)

        def _dot(a, b):
            return jax.lax.dot_general(
                a, b, (((1,), (1,)), ((), ())),
                preferred_element_type=jnp.float32,
            )

        g8_ref[hh] += _dot(qhi, khi) + _dot(qhi, klo) + _dot(qlo, khi)
        s2_ref[hh] += kblk * kblk
        q2_ref[...] += qblk * qblk

    @pl.when(pp == NFULL)
    def _tail():
        dtail = jnp.zeros((Q, CH), jnp.float32)
        s2t = jnp.zeros((CH, D), jnp.float32)
        q2t = jnp.zeros((Q, D), jnp.float32)
        for p in range(TAIL):
            qp = q_ref[p, :, :]
            kp = k_ref[:, p, :]
            dtail += jax.lax.dot_general(
                qp, kp, (((1,), (1,)), ((), ())),
                preferred_element_type=jnp.float32,
                precision=jax.lax.Precision.HIGHEST,
            )
            s2t += kp * kp
            q2t += qp * qp
        dt_ref[hh] = dtail
        s2t_ref[hh] = s2t

        @pl.when(hh == 1)
        def _fin():
            row = jax.lax.broadcasted_iota(jnp.int32, (QR, CRH), 0)
            col = jax.lax.broadcasted_iota(jnp.int32, (QR, CRH), 1)
            diag = (row // Q) == (col % PB)
            srow = jax.lax.broadcasted_iota(jnp.int32, (Q, QR), 0)
            scol = jax.lax.broadcasted_iota(jnp.int32, (Q, QR), 1)
            s_fold = jnp.where(srow == scol % Q, 1.0, 0.0)
            frow = jax.lax.broadcasted_iota(jnp.int32, (CRH, CH), 0)
            fcol = jax.lax.broadcasted_iota(jnp.int32, (CRH, CH), 1)
            f_fold = jnp.where(frow // PB == fcol, 1.0, 0.0)

            def _extract(h):
                g8m = jnp.where(diag, g8_ref[h], 0.0)
                gq = jax.lax.dot_general(
                    s_fold, g8m, (((1,), (0,)), ((), ())),
                    preferred_element_type=jnp.float32,
                    precision=jax.lax.Precision.HIGHEST,
                )
                return jax.lax.dot_general(
                    gq, f_fold, (((1,), (0,)), ((), ())),
                    preferred_element_type=jnp.float32,
                    precision=jax.lax.Precision.HIGHEST,
                ) + dt_ref[h]

            dot = jnp.concatenate([_extract(0), _extract(1)], axis=1)
            ks = jnp.concatenate(
                [jnp.sum(s2_ref[0], axis=(1, 2)) + jnp.sum(s2t_ref[0], axis=1),
                 jnp.sum(s2_ref[1], axis=(1, 2)) + jnp.sum(s2t_ref[1], axis=1)])
            qs = (0.5 * jnp.sum(q2_ref[...], axis=(0, 2))
                  + jnp.sum(q2t, axis=1))[:, None]
            d2 = jnp.maximum(qs + ks[None, :] - 2.0 * dot, 0.0)
            idx_ref[...] = jnp.argmin(d2, axis=1).astype(jnp.int32)
            dist_ref[...] = jnp.sqrt(jnp.min(d2, axis=1))


def kernel(query_features, keys):
    qt = jnp.swapaxes(query_features, 0, 1)
    dist, idx = pl.pallas_call(
        _body,
        grid=(NFULL + 1, 2),
        in_specs=[
            pl.BlockSpec((PB, Q, D), lambda p, h: (p, 0, 0)),
            pl.BlockSpec((CH, PB, D), lambda p, h: (h, p, 0)),
        ],
        out_specs=[
            pl.BlockSpec((Q,), lambda p, h: (0,)),
            pl.BlockSpec((Q,), lambda p, h: (0,)),
        ],
        out_shape=[
            jax.ShapeDtypeStruct((Q,), jnp.float32),
            jax.ShapeDtypeStruct((Q,), jnp.int32),
        ],
        scratch_shapes=[
            pltpu.VMEM((2, QR, CRH), jnp.float32),
            pltpu.VMEM((2, CH, PB, D), jnp.float32),
            pltpu.VMEM((PB, Q, D), jnp.float32),
            pltpu.VMEM((2, Q, CH), jnp.float32),
            pltpu.VMEM((2, CH, D), jnp.float32),
        ],
    )(qt, keys)
    return dist, idx
